# baseline (speedup 1.0000x reference)
import functools

import jax
import jax.numpy as jnp
from jax import lax
from jax.experimental import pallas as pl
from jax.experimental.pallas import tpu as pltpu
from jax.experimental.pallas import tpu_sc as plsc

_NW = 32


def _sc_gather3(ntab, ttab, emb, idx):
    n_idx = idx.shape[0]
    nn = ntab.shape[1]
    d = emb.shape[1]
    b = n_idx // _NW
    mesh = plsc.VectorSubcoreMesh(core_axis_name="c", subcore_axis_name="s")

    @functools.partial(
        pl.kernel,
        out_type=(
            jax.ShapeDtypeStruct((n_idx, nn), jnp.int32),
            jax.ShapeDtypeStruct((n_idx, nn), jnp.float32),
            jax.ShapeDtypeStruct((n_idx, d), jnp.float32),
        ),
        mesh=mesh,
        compiler_params=pltpu.CompilerParams(use_tc_tiling_on_sc=False),
        scratch_types=[
            pltpu.VMEM((b,), jnp.int32),
            pltpu.VMEM((b, nn), jnp.int32),
            pltpu.VMEM((b, nn), jnp.float32),
            pltpu.VMEM((b, d), jnp.float32),
            pltpu.SemaphoreType.DMA,
            pltpu.SemaphoreType.DMA,
            pltpu.SemaphoreType.DMA,
        ],
    )
    def k(ntab_hbm, ttab_hbm, emb_hbm, idx_hbm, n_out, t_out, e_out,
          idx_v, n_v, t_v, e_v, sem1, sem2, sem3):
        wid = lax.axis_index("s") * 2 + lax.axis_index("c")
        base = wid * b
        pltpu.sync_copy(idx_hbm.at[pl.ds(base, b)], idx_v)
        c1 = pltpu.async_copy(ntab_hbm.at[idx_v], n_v, sem1)
        c2 = pltpu.async_copy(ttab_hbm.at[idx_v], t_v, sem2)
        c3 = pltpu.async_copy(emb_hbm.at[idx_v], e_v, sem3)
        c1.wait()
        c2.wait()
        c3.wait()
        pltpu.sync_copy(n_v, n_out.at[pl.ds(base, b)])
        pltpu.sync_copy(t_v, t_out.at[pl.ds(base, b)])
        pltpu.sync_copy(e_v, e_out.at[pl.ds(base, b)])

    return k(ntab, ttab, emb, idx)


def _sc_gather_embed(emb, idx, chunk):
    n_idx = idx.shape[0]
    d = emb.shape[1]
    b = n_idx // _NW
    nchunks = b // chunk
    mesh = plsc.VectorSubcoreMesh(core_axis_name="c", subcore_axis_name="s")

    @functools.partial(
        pl.kernel,
        out_type=jax.ShapeDtypeStruct((n_idx, d), jnp.float32),
        mesh=mesh,
        scratch_types=[
            pltpu.VMEM((chunk,), jnp.int32),
            pltpu.VMEM((chunk, d), jnp.float32),
            pltpu.SemaphoreType.DMA,
        ],
    )
    def k(emb_hbm, idx_hbm, out_hbm, idx_v, rows_v, sem):
        wid = lax.axis_index("s") * 2 + lax.axis_index("c")
        base = wid * b
        for c in range(nchunks):
            off = base + c * chunk
            pltpu.sync_copy(idx_hbm.at[pl.ds(off, chunk)], idx_v)
            pltpu.async_copy(emb_hbm.at[idx_v], rows_v, sem).wait()
            pltpu.sync_copy(rows_v, out_hbm.at[pl.ds(off, chunk)])

    return k(emb, idx)


def _fast_cos(x):
    pi2_hi = 6.2831854820251465
    pi2_lo = -1.7484555314695172e-07
    k = jnp.floor(x * 0.15915494309189535 + 0.5)
    y = (x - k * pi2_hi) - k * pi2_lo
    u = y * y
    acc = jnp.float32(-1.1470745597729725e-11)
    for c in (2.08767569878681e-09, -2.755731922398589e-07,
              2.48015873015873e-05, -1.3888888888888889e-03,
              4.1666666666666664e-02, -0.5, 1.0):
        acc = acc * u + c
    return acc


def _attn_body(src_ref, seq_ref, dt_ref, ngh_ref, freq_ref, phase_ref,
               wq_ref, wk_ref, wv_ref, wo_ref, mw1_ref, mb1_ref, mw2_ref,
               mb2_ref, out_ref):
    bq, d = src_ref.shape
    rows = seq_ref.shape[0]
    nn = rows // bq
    m = wq_ref.shape[0]
    n_head = 4
    d_head = m // n_head

    src = src_ref[...]
    phase = phase_ref[...]
    src_t = jnp.cos(phase)
    q_in = jnp.concatenate(
        [src, lax.broadcast_in_dim(src_t, (bq, d), (0, 1))], axis=1)
    q = jnp.dot(q_in, wq_ref[...], preferred_element_type=jnp.float32)

    dt = dt_ref[...]
    arg = (lax.broadcast_in_dim(dt, (rows, d), (0, 1))
           * lax.broadcast_in_dim(freq_ref[...], (rows, d), (0, 1))
           + lax.broadcast_in_dim(phase, (rows, d), (0, 1)))
    t_enc = jnp.cos(arg)
    k_in = jnp.concatenate([seq_ref[...], t_enc], axis=1)
    kk = jnp.dot(k_in, wk_ref[...], preferred_element_type=jnp.float32)
    vv = jnp.dot(k_in, wv_ref[...], preferred_element_type=jnp.float32)

    qe = lax.broadcast_in_dim(q, (bq, nn, m), (0, 2)).reshape(rows, m)
    prod = qe * kk
    mask3 = (ngh_ref[...] == 0).reshape(bq, nn, 1)
    scale = 1.0 / (d_head ** 0.5)
    a_cols = []
    for h in range(n_head):
        sh = jnp.sum(prod[:, h * d_head:(h + 1) * d_head], axis=1,
                     keepdims=True)
        sh = sh.reshape(bq, nn, 1) * scale
        sh = jnp.where(mask3, -1e10, sh)
        mh = jnp.max(sh, axis=1, keepdims=True)
        eh = jnp.exp(sh - mh)
        ah = eh / jnp.sum(eh, axis=1, keepdims=True)
        a_cols.append(
            lax.broadcast_in_dim(ah, (bq, nn, d_head), (0, 1, 2))
            .reshape(rows, d_head))
    a2 = jnp.concatenate(a_cols, axis=1)
    o = jnp.sum((a2 * vv).reshape(bq, nn, m), axis=1)
    o = jnp.dot(o, wo_ref[...], preferred_element_type=jnp.float32)

    h1 = jnp.concatenate([o, src], axis=1)
    h1 = jnp.maximum(
        jnp.dot(h1, mw1_ref[...], preferred_element_type=jnp.float32)
        + mb1_ref[...], 0.0)
    out_ref[...] = (jnp.dot(h1, mw2_ref[...], preferred_element_type=jnp.float32)
                    + mb2_ref[...])


def _attn_layer(src, seq_flat, dt_flat, ngh_flat, freq2, phase2, p, bq):
    nq, d = src.shape
    nn = seq_flat.shape[0] // nq
    m = 2 * d
    grid = (nq // bq,)
    full = lambda r, c: pl.BlockSpec((r, c), lambda i: (0, 0))
    return pl.pallas_call(
        _attn_body,
        grid=grid,
        in_specs=[
            pl.BlockSpec((bq, d), lambda i: (i, 0)),
            pl.BlockSpec((bq * nn, d), lambda i: (i, 0)),
            pl.BlockSpec((bq * nn, 1), lambda i: (i, 0)),
            pl.BlockSpec((bq * nn, 1), lambda i: (i, 0)),
            full(1, d),
            full(1, d),
            full(m, m),
            full(m, m),
            full(m, m),
            full(m, m),
            full(m + d, d),
            full(1, d),
            full(d, d),
            full(1, d),
        ],
        out_specs=pl.BlockSpec((bq, d), lambda i: (i, 0)),
        out_shape=jax.ShapeDtypeStruct((nq, d), jnp.float32),
    )(src, seq_flat, dt_flat, ngh_flat, freq2, phase2,
      p['Wq'], p['Wk'], p['Wv'], p['Wo'],
      p['mw1'], p['mb1'][None, :], p['mw2'], p['mb2'][None, :])


def _score_body(cs_ref, ct_ref, aw1_ref, ab1_ref, aw2_ref, ab2_ref, out_ref):
    hcat = jnp.concatenate([cs_ref[...], ct_ref[...]], axis=1)
    h = jnp.maximum(
        jnp.dot(hcat, aw1_ref[...], preferred_element_type=jnp.float32)
        + ab1_ref[...], 0.0)
    out_ref[...] = (jnp.dot(h, aw2_ref[...], preferred_element_type=jnp.float32)
                    + ab2_ref[...])


def _score_call(cs, ct, aw1, ab1, aw2, ab2):
    b, d = cs.shape
    aw2p = jnp.pad(aw2, ((0, 0), (0, d - aw2.shape[1])))
    ab2p = jnp.pad(ab2, (0, d - ab2.shape[0]))[None, :]
    out = pl.pallas_call(
        _score_body,
        out_shape=jax.ShapeDtypeStruct((b, d), jnp.float32),
    )(cs, ct, aw1, ab1[None, :], aw2p, ab2p)
    return out[:, 0]


def kernel(src_idx_l, target_idx_l, cut_time_l, num_neighbors, node_embed,
           ngh_node_table, ngh_time_table, time_freq, time_phase, params):
    nn = ngh_node_table.shape[1]

    q1 = jnp.concatenate([src_idx_l, target_idx_l])
    t1 = jnp.concatenate([cut_time_l, cut_time_l])

    n1, tn1, e_q1 = _sc_gather3(ngh_node_table, ngh_time_table, node_embed, q1)
    q2 = n1.reshape(-1)
    t2 = tn1.reshape(-1)
    n2, tn2, e_q2 = _sc_gather3(ngh_node_table, ngh_time_table, node_embed, q2)
    qk = n2.reshape(-1)
    ek2 = _sc_gather_embed(node_embed, qk, chunk=512)

    colpad = jnp.arange(nn)[None, :] >= num_neighbors
    n1m = jnp.where(colpad, 0, n1)
    n2m = jnp.where(colpad, 0, n2)

    freq2 = time_freq[None, :]
    phase2 = time_phase[None, :]

    dt1 = (t1[:, None] - tn1).reshape(-1, 1)
    dt2 = (t2[:, None] - tn2).reshape(-1, 1)
    ngh1 = n1m.reshape(-1, 1)
    ngh2 = n2m.reshape(-1, 1)

    c1a = _attn_layer(e_q1, e_q2, dt1, ngh1,
                      freq2, phase2, params['layer0'], bq=128)
    c1b = _attn_layer(e_q2, ek2, dt2, ngh2,
                      freq2, phase2, params['layer0'], bq=128)
    c2 = _attn_layer(c1a, c1b, dt1, ngh1,
                     freq2, phase2, params['layer1'], bq=128)

    b = src_idx_l.shape[0]
    return _score_call(c2[:b], c2[b:], params['aw1'], params['ab1'],
                       params['aw2'], params['ab2'])


# speedup vs baseline: 1.5860x; 1.5860x over previous
import functools

import jax
import jax.numpy as jnp
from jax import lax
from jax.experimental import pallas as pl
from jax.experimental.pallas import tpu as pltpu
from jax.experimental.pallas import tpu_sc as plsc

_NW = 32


def _sc_gather3(ntab, ttab, emb, idx):
    n_idx = idx.shape[0]
    nn = ntab.shape[1]
    d = emb.shape[1]
    b = n_idx // _NW
    mesh = plsc.VectorSubcoreMesh(core_axis_name="c", subcore_axis_name="s")

    @functools.partial(
        pl.kernel,
        out_type=(
            jax.ShapeDtypeStruct((n_idx, nn), jnp.int32),
            jax.ShapeDtypeStruct((n_idx, nn), jnp.float32),
            jax.ShapeDtypeStruct((n_idx, d), jnp.float32),
        ),
        mesh=mesh,
        compiler_params=pltpu.CompilerParams(use_tc_tiling_on_sc=False),
        scratch_types=[
            pltpu.VMEM((b,), jnp.int32),
            pltpu.VMEM((b, nn), jnp.int32),
            pltpu.VMEM((b, nn), jnp.float32),
            pltpu.VMEM((b, d), jnp.float32),
            pltpu.SemaphoreType.DMA,
            pltpu.SemaphoreType.DMA,
            pltpu.SemaphoreType.DMA,
        ],
    )
    def k(ntab_hbm, ttab_hbm, emb_hbm, idx_hbm, n_out, t_out, e_out,
          idx_v, n_v, t_v, e_v, sem1, sem2, sem3):
        wid = lax.axis_index("s") * 2 + lax.axis_index("c")
        base = wid * b
        pltpu.sync_copy(idx_hbm.at[pl.ds(base, b)], idx_v)
        c1 = pltpu.async_copy(ntab_hbm.at[idx_v], n_v, sem1)
        c2 = pltpu.async_copy(ttab_hbm.at[idx_v], t_v, sem2)
        c3 = pltpu.async_copy(emb_hbm.at[idx_v], e_v, sem3)
        c1.wait()
        c2.wait()
        c3.wait()
        pltpu.sync_copy(n_v, n_out.at[pl.ds(base, b)])
        pltpu.sync_copy(t_v, t_out.at[pl.ds(base, b)])
        pltpu.sync_copy(e_v, e_out.at[pl.ds(base, b)])

    return k(ntab, ttab, emb, idx)


def _sc_gather_embed(emb, idx, chunk):
    n_idx = idx.shape[0]
    d = emb.shape[1]
    b = n_idx // _NW
    nchunks = b // chunk
    mesh = plsc.VectorSubcoreMesh(core_axis_name="c", subcore_axis_name="s")

    @functools.partial(
        pl.kernel,
        out_type=jax.ShapeDtypeStruct((n_idx, d), jnp.float32),
        mesh=mesh,
        scratch_types=[
            pltpu.VMEM((chunk,), jnp.int32),
            pltpu.VMEM((chunk, d), jnp.float32),
            pltpu.SemaphoreType.DMA,
        ],
    )
    def k(emb_hbm, idx_hbm, out_hbm, idx_v, rows_v, sem):
        wid = lax.axis_index("s") * 2 + lax.axis_index("c")
        base = wid * b
        for c in range(nchunks):
            off = base + c * chunk
            pltpu.sync_copy(idx_hbm.at[pl.ds(off, chunk)], idx_v)
            pltpu.async_copy(emb_hbm.at[idx_v], rows_v, sem).wait()
            pltpu.sync_copy(rows_v, out_hbm.at[pl.ds(off, chunk)])

    return k(emb, idx)


def _precise_cos(x):
    k = jnp.floor(x * 0.3183098861837907 + 0.5)
    y = ((x - k * 3.140625) - k * 0.0009676535846665502) \
        - k * 5.126565838509123e-12
    u = y * y
    acc = jnp.float32(-1.1470745597729725e-11)
    for c in (2.08767569878681e-09, -2.755731922398589e-07,
              2.48015873015873e-05, -1.3888888888888889e-03,
              4.1666666666666664e-02, -0.5, 1.0):
        acc = acc * u + c
    parity = k - 2.0 * jnp.floor(k * 0.5)
    return acc * (1.0 - 2.0 * parity)


def _attn_body(src_ref, seq_ref, t_ref, tn_ref, ngh_ref, freq_ref, phase_ref,
               wq_ref, wk_ref, wv_ref, wo_ref, mw1_ref, mb1_ref, mw2_ref,
               mb2_ref, out_ref):
    bq, d = src_ref.shape
    nn = tn_ref.shape[1]
    m = wq_ref.shape[0]
    n_head = 4
    d_head = m // n_head

    src = src_ref[...]
    phase = phase_ref[...]
    src_t = jnp.cos(phase)
    q_in = jnp.concatenate(
        [src, lax.broadcast_in_dim(src_t, (bq, d), (0, 1))], axis=1)
    q = jnp.dot(q_in, wq_ref[...], preferred_element_type=jnp.float32)

    delta = t_ref[...] - tn_ref[...]
    d3 = lax.broadcast_in_dim(delta, (bq, nn, d), (0, 1))
    freq3 = lax.broadcast_in_dim(freq_ref[...], (bq, nn, d), (1, 2))
    phase3 = lax.broadcast_in_dim(phase, (bq, nn, d), (1, 2))
    t_enc = _precise_cos(d3 * freq3 + phase3)
    seq = seq_ref[...].reshape(bq, nn, d)
    k_in = jnp.concatenate([seq, t_enc], axis=2).reshape(bq * nn, m)
    kk = jnp.dot(k_in, wk_ref[...], preferred_element_type=jnp.float32)
    vv = jnp.dot(k_in, wv_ref[...], preferred_element_type=jnp.float32)

    prod = lax.broadcast_in_dim(q, (bq, nn, m), (0, 2)) * kk.reshape(bq, nn, m)
    vv3 = vv.reshape(bq, nn, m)
    mask = ngh_ref[...] == 0
    scale = 1.0 / (d_head ** 0.5)
    o_heads = []
    for h in range(n_head):
        sh = jnp.sum(prod[:, :, h * d_head:(h + 1) * d_head], axis=2) * scale
        sh = jnp.where(mask, -1e10, sh)
        mh = jnp.max(sh, axis=1, keepdims=True)
        eh = jnp.exp(sh - mh)
        ah = eh / jnp.sum(eh, axis=1, keepdims=True)
        vh = vv3[:, :, h * d_head:(h + 1) * d_head]
        oh = jnp.sum(lax.broadcast_in_dim(ah, (bq, nn, d_head), (0, 1)) * vh,
                     axis=1)
        o_heads.append(oh)
    o = jnp.concatenate(o_heads, axis=1)
    o = jnp.dot(o, wo_ref[...], preferred_element_type=jnp.float32)

    h1 = jnp.concatenate([o, src], axis=1)
    h1 = jnp.maximum(
        jnp.dot(h1, mw1_ref[...], preferred_element_type=jnp.float32)
        + mb1_ref[...], 0.0)
    out_ref[...] = (jnp.dot(h1, mw2_ref[...], preferred_element_type=jnp.float32)
                    + mb2_ref[...])


def _attn_layer(src, seq_flat, t, tn, ngh, freq2, phase2, p, bq):
    nq, d = src.shape
    nn = tn.shape[1]
    m = 2 * d
    grid = (nq // bq,)
    full = lambda r, c: pl.BlockSpec((r, c), lambda i: (0, 0))
    return pl.pallas_call(
        _attn_body,
        grid=grid,
        in_specs=[
            pl.BlockSpec((bq, d), lambda i: (i, 0)),
            pl.BlockSpec((bq * nn, d), lambda i: (i, 0)),
            pl.BlockSpec((bq, 1), lambda i: (i, 0)),
            pl.BlockSpec((bq, nn), lambda i: (i, 0)),
            pl.BlockSpec((bq, nn), lambda i: (i, 0)),
            full(1, d),
            full(1, d),
            full(m, m),
            full(m, m),
            full(m, m),
            full(m, m),
            full(m + d, d),
            full(1, d),
            full(d, d),
            full(1, d),
        ],
        out_specs=pl.BlockSpec((bq, d), lambda i: (i, 0)),
        out_shape=jax.ShapeDtypeStruct((nq, d), jnp.float32),
    )(src, seq_flat, t, tn, ngh, freq2, phase2,
      p['Wq'], p['Wk'], p['Wv'], p['Wo'],
      p['mw1'], p['mb1'][None, :], p['mw2'], p['mb2'][None, :])


def _score_body(cs_ref, ct_ref, aw1_ref, ab1_ref, aw2_ref, ab2_ref, out_ref):
    hcat = jnp.concatenate([cs_ref[...], ct_ref[...]], axis=1)
    h = jnp.maximum(
        jnp.dot(hcat, aw1_ref[...], preferred_element_type=jnp.float32)
        + ab1_ref[...], 0.0)
    out_ref[...] = (jnp.dot(h, aw2_ref[...], preferred_element_type=jnp.float32)
                    + ab2_ref[...])


def _score_call(cs, ct, aw1, ab1, aw2, ab2):
    b, d = cs.shape
    aw2p = jnp.pad(aw2, ((0, 0), (0, d - aw2.shape[1])))
    ab2p = jnp.pad(ab2, (0, d - ab2.shape[0]))[None, :]
    out = pl.pallas_call(
        _score_body,
        out_shape=jax.ShapeDtypeStruct((b, d), jnp.float32),
    )(cs, ct, aw1, ab1[None, :], aw2p, ab2p)
    return out[:, 0]


def kernel(src_idx_l, target_idx_l, cut_time_l, num_neighbors, node_embed,
           ngh_node_table, ngh_time_table, time_freq, time_phase, params):
    nn = ngh_node_table.shape[1]

    q1 = jnp.concatenate([src_idx_l, target_idx_l])
    t1 = jnp.concatenate([cut_time_l, cut_time_l])

    n1, tn1, e_q1 = _sc_gather3(ngh_node_table, ngh_time_table, node_embed, q1)
    q2 = n1.reshape(-1)
    t2 = tn1.reshape(-1)
    n2, tn2, e_q2 = _sc_gather3(ngh_node_table, ngh_time_table, node_embed, q2)
    qk = n2.reshape(-1)
    ek2 = _sc_gather_embed(node_embed, qk, chunk=512)

    colpad = jnp.arange(nn)[None, :] >= num_neighbors
    n1m = jnp.where(colpad, 0, n1)
    n2m = jnp.where(colpad, 0, n2)

    freq2 = time_freq[None, :]
    phase2 = time_phase[None, :]

    c1a = _attn_layer(e_q1, e_q2, t1[:, None], tn1, n1m,
                      freq2, phase2, params['layer0'], bq=128)
    c1b = _attn_layer(e_q2, ek2, t2[:, None], tn2, n2m,
                      freq2, phase2, params['layer0'], bq=128)
    c2 = _attn_layer(c1a, c1b, t1[:, None], tn1, n1m,
                     freq2, phase2, params['layer1'], bq=128)

    b = src_idx_l.shape[0]
    return _score_call(c2[:b], c2[b:], params['aw1'], params['ab1'],
                       params['aw2'], params['ab2'])


# speedup vs baseline: 1.6818x; 1.0604x over previous
import functools

import jax
import jax.numpy as jnp
from jax import lax
from jax.experimental import pallas as pl
from jax.experimental.pallas import tpu as pltpu
from jax.experimental.pallas import tpu_sc as plsc

_NW = 32


def _sc_gather3(ntab, ttab, emb, idx):
    n_idx = idx.shape[0]
    nn = ntab.shape[1]
    d = emb.shape[1]
    b = n_idx // _NW
    mesh = plsc.VectorSubcoreMesh(core_axis_name="c", subcore_axis_name="s")

    @functools.partial(
        pl.kernel,
        out_type=(
            jax.ShapeDtypeStruct((n_idx, nn), jnp.int32),
            jax.ShapeDtypeStruct((n_idx, nn), jnp.float32),
            jax.ShapeDtypeStruct((n_idx, d), jnp.float32),
        ),
        mesh=mesh,
        compiler_params=pltpu.CompilerParams(use_tc_tiling_on_sc=False),
        scratch_types=[
            pltpu.VMEM((b,), jnp.int32),
            pltpu.VMEM((b, nn), jnp.int32),
            pltpu.VMEM((b, nn), jnp.float32),
            pltpu.VMEM((b, d), jnp.float32),
            pltpu.SemaphoreType.DMA,
            pltpu.SemaphoreType.DMA,
            pltpu.SemaphoreType.DMA,
        ],
    )
    def k(ntab_hbm, ttab_hbm, emb_hbm, idx_hbm, n_out, t_out, e_out,
          idx_v, n_v, t_v, e_v, sem1, sem2, sem3):
        wid = lax.axis_index("s") * 2 + lax.axis_index("c")
        base = wid * b
        pltpu.sync_copy(idx_hbm.at[pl.ds(base, b)], idx_v)
        c1 = pltpu.async_copy(ntab_hbm.at[idx_v], n_v, sem1)
        c2 = pltpu.async_copy(ttab_hbm.at[idx_v], t_v, sem2)
        c3 = pltpu.async_copy(emb_hbm.at[idx_v], e_v, sem3)
        c1.wait()
        c2.wait()
        c3.wait()
        pltpu.sync_copy(n_v, n_out.at[pl.ds(base, b)])
        pltpu.sync_copy(t_v, t_out.at[pl.ds(base, b)])
        pltpu.sync_copy(e_v, e_out.at[pl.ds(base, b)])

    return k(ntab, ttab, emb, idx)


def _sc_gather_embed(emb, idx, chunk):
    n_idx = idx.shape[0]
    d = emb.shape[1]
    b = n_idx // _NW
    nchunks = b // chunk
    mesh = plsc.VectorSubcoreMesh(core_axis_name="c", subcore_axis_name="s")

    @functools.partial(
        pl.kernel,
        out_type=jax.ShapeDtypeStruct((n_idx, d), jnp.float32),
        mesh=mesh,
        scratch_types=[
            pltpu.VMEM((chunk,), jnp.int32),
            pltpu.VMEM((chunk, d), jnp.float32),
            pltpu.SemaphoreType.DMA,
        ],
    )
    def k(emb_hbm, idx_hbm, out_hbm, idx_v, rows_v, sem):
        wid = lax.axis_index("s") * 2 + lax.axis_index("c")
        base = wid * b
        for c in range(nchunks):
            off = base + c * chunk
            pltpu.sync_copy(idx_hbm.at[pl.ds(off, chunk)], idx_v)
            pltpu.async_copy(emb_hbm.at[idx_v], rows_v, sem).wait()
            pltpu.sync_copy(rows_v, out_hbm.at[pl.ds(off, chunk)])

    return k(emb, idx)


def _precise_cos(x):
    k = jnp.floor(x * 0.3183098861837907 + 0.5)
    y = ((x - k * 3.140625) - k * 0.0009676535846665502) \
        - k * 5.126565838509123e-12
    u = y * y
    acc = jnp.float32(-1.1470745597729725e-11)
    for c in (2.08767569878681e-09, -2.755731922398589e-07,
              2.48015873015873e-05, -1.3888888888888889e-03,
              4.1666666666666664e-02, -0.5, 1.0):
        acc = acc * u + c
    parity = k - 2.0 * jnp.floor(k * 0.5)
    return acc * (1.0 - 2.0 * parity)


def _attn_body(src_ref, seq_ref, t_ref, tn_ref, ngh_ref, freq_ref, phase_ref,
               wq_ref, wk_ref, wv_ref, wo_ref, mw1_ref, mb1_ref, mw2_ref,
               mb2_ref, out_ref):
    bq, d = src_ref.shape
    nn = tn_ref.shape[1]
    m = wq_ref.shape[0]
    n_head = 4
    d_head = m // n_head

    src = src_ref[...]
    phase = phase_ref[...]
    src_t = jnp.cos(phase)
    q_in = jnp.concatenate(
        [src, lax.broadcast_in_dim(src_t, (bq, d), (0, 1))], axis=1)
    q = jnp.dot(q_in, wq_ref[...], preferred_element_type=jnp.float32)

    delta = t_ref[...] - tn_ref[...]
    d3 = lax.broadcast_in_dim(delta, (bq, nn, d), (0, 1))
    freq3 = lax.broadcast_in_dim(freq_ref[...], (bq, nn, d), (1, 2))
    phase3 = lax.broadcast_in_dim(phase, (bq, nn, d), (1, 2))
    t_enc = _precise_cos(d3 * freq3 + phase3)
    seq = seq_ref[...].reshape(bq, nn, d)
    k_in = jnp.concatenate([seq, t_enc], axis=2).reshape(bq * nn, m)
    kk = jnp.dot(k_in, wk_ref[...], preferred_element_type=jnp.float32)
    vv = jnp.dot(k_in, wv_ref[...], preferred_element_type=jnp.float32)

    prod = lax.broadcast_in_dim(q, (bq, nn, m), (0, 2)) * kk.reshape(bq, nn, m)
    vv3 = vv.reshape(bq, nn, m)
    mask = ngh_ref[...] == 0
    scale = 1.0 / (d_head ** 0.5)
    o_heads = []
    for h in range(n_head):
        sh = jnp.sum(prod[:, :, h * d_head:(h + 1) * d_head], axis=2) * scale
        sh = jnp.where(mask, -1e10, sh)
        mh = jnp.max(sh, axis=1, keepdims=True)
        eh = jnp.exp(sh - mh)
        ah = eh / jnp.sum(eh, axis=1, keepdims=True)
        vh = vv3[:, :, h * d_head:(h + 1) * d_head]
        oh = jnp.sum(lax.broadcast_in_dim(ah, (bq, nn, d_head), (0, 1)) * vh,
                     axis=1)
        o_heads.append(oh)
    o = jnp.concatenate(o_heads, axis=1)
    o = jnp.dot(o, wo_ref[...], preferred_element_type=jnp.float32)

    h1 = jnp.concatenate([o, src], axis=1)
    h1 = jnp.maximum(
        jnp.dot(h1, mw1_ref[...], preferred_element_type=jnp.float32)
        + mb1_ref[...], 0.0)
    out_ref[...] = (jnp.dot(h1, mw2_ref[...], preferred_element_type=jnp.float32)
                    + mb2_ref[...])


def _attn_layer(src, seq_flat, t, tn, ngh, freq2, phase2, p, bq):
    nq, d = src.shape
    nn = tn.shape[1]
    m = 2 * d
    grid = (nq // bq,)
    full = lambda r, c: pl.BlockSpec((r, c), lambda i: (0, 0))
    return pl.pallas_call(
        _attn_body,
        grid=grid,
        in_specs=[
            pl.BlockSpec((bq, d), lambda i: (i, 0)),
            pl.BlockSpec((bq * nn, d), lambda i: (i, 0)),
            pl.BlockSpec((bq, 1), lambda i: (i, 0)),
            pl.BlockSpec((bq, nn), lambda i: (i, 0)),
            pl.BlockSpec((bq, nn), lambda i: (i, 0)),
            full(1, d),
            full(1, d),
            full(m, m),
            full(m, m),
            full(m, m),
            full(m, m),
            full(m + d, d),
            full(1, d),
            full(d, d),
            full(1, d),
        ],
        out_specs=pl.BlockSpec((bq, d), lambda i: (i, 0)),
        out_shape=jax.ShapeDtypeStruct((nq, d), jnp.float32),
    )(src, seq_flat, t, tn, ngh, freq2, phase2,
      p['Wq'], p['Wk'], p['Wv'], p['Wo'],
      p['mw1'], p['mb1'][None, :], p['mw2'], p['mb2'][None, :])


def _score_body(cs_ref, ct_ref, aw1_ref, ab1_ref, aw2_ref, ab2_ref, out_ref):
    hcat = jnp.concatenate([cs_ref[...], ct_ref[...]], axis=1)
    h = jnp.maximum(
        jnp.dot(hcat, aw1_ref[...], preferred_element_type=jnp.float32)
        + ab1_ref[...], 0.0)
    out_ref[...] = (jnp.dot(h, aw2_ref[...], preferred_element_type=jnp.float32)
                    + ab2_ref[...])


def _score_call(cs, ct, aw1, ab1, aw2, ab2):
    b, d = cs.shape
    aw2p = jnp.pad(aw2, ((0, 0), (0, d - aw2.shape[1])))
    ab2p = jnp.pad(ab2, (0, d - ab2.shape[0]))[None, :]
    out = pl.pallas_call(
        _score_body,
        out_shape=jax.ShapeDtypeStruct((b, d), jnp.float32),
    )(cs, ct, aw1, ab1[None, :], aw2p, ab2p)
    return out[:, 0]


def kernel(src_idx_l, target_idx_l, cut_time_l, num_neighbors, node_embed,
           ngh_node_table, ngh_time_table, time_freq, time_phase, params):
    nn = ngh_node_table.shape[1]

    q1 = jnp.concatenate([src_idx_l, target_idx_l])
    t1 = jnp.concatenate([cut_time_l, cut_time_l])

    n1, tn1, e_q1 = _sc_gather3(ngh_node_table, ngh_time_table, node_embed, q1)
    q2 = n1.reshape(-1)
    t2 = tn1.reshape(-1)
    n2, tn2, e_q2 = _sc_gather3(ngh_node_table, ngh_time_table, node_embed, q2)
    qk = n2.reshape(-1)
    ek2 = _sc_gather_embed(node_embed, qk, chunk=512)

    colpad = jnp.arange(nn)[None, :] >= num_neighbors
    n1m = jnp.where(colpad, 0, n1)
    n2m = jnp.where(colpad, 0, n2)

    freq2 = time_freq[None, :]
    phase2 = time_phase[None, :]

    c1a = _attn_layer(e_q1, e_q2, t1[:, None], tn1, n1m,
                      freq2, phase2, params['layer0'], bq=256)
    c1b = _attn_layer(e_q2, ek2, t2[:, None], tn2, n2m,
                      freq2, phase2, params['layer0'], bq=256)
    c2 = _attn_layer(c1a, c1b, t1[:, None], tn1, n1m,
                     freq2, phase2, params['layer1'], bq=256)

    b = src_idx_l.shape[0]
    return _score_call(c2[:b], c2[b:], params['aw1'], params['ab1'],
                       params['aw2'], params['ab2'])


# speedup vs baseline: 1.8923x; 1.1251x over previous
import functools

import jax
import jax.numpy as jnp
from jax import lax
from jax.experimental import pallas as pl
from jax.experimental.pallas import tpu as pltpu
from jax.experimental.pallas import tpu_sc as plsc

_NW = 32


def _sc_gather3(ntab, ttab, emb, idx):
    n_idx = idx.shape[0]
    nn = ntab.shape[1]
    d = emb.shape[1]
    b = n_idx // _NW
    mesh = plsc.VectorSubcoreMesh(core_axis_name="c", subcore_axis_name="s")

    @functools.partial(
        pl.kernel,
        out_type=(
            jax.ShapeDtypeStruct((n_idx, nn), jnp.int32),
            jax.ShapeDtypeStruct((n_idx, nn), jnp.float32),
            jax.ShapeDtypeStruct((n_idx, d), jnp.float32),
        ),
        mesh=mesh,
        compiler_params=pltpu.CompilerParams(use_tc_tiling_on_sc=False),
        scratch_types=[
            pltpu.VMEM((b,), jnp.int32),
            pltpu.VMEM((b, nn), jnp.int32),
            pltpu.VMEM((b, nn), jnp.float32),
            pltpu.VMEM((b, d), jnp.float32),
            pltpu.SemaphoreType.DMA,
            pltpu.SemaphoreType.DMA,
            pltpu.SemaphoreType.DMA,
        ],
    )
    def k(ntab_hbm, ttab_hbm, emb_hbm, idx_hbm, n_out, t_out, e_out,
          idx_v, n_v, t_v, e_v, sem1, sem2, sem3):
        wid = lax.axis_index("s") * 2 + lax.axis_index("c")
        base = wid * b
        pltpu.sync_copy(idx_hbm.at[pl.ds(base, b)], idx_v)
        c1 = pltpu.async_copy(ntab_hbm.at[idx_v], n_v, sem1)
        c2 = pltpu.async_copy(ttab_hbm.at[idx_v], t_v, sem2)
        c3 = pltpu.async_copy(emb_hbm.at[idx_v], e_v, sem3)
        c1.wait()
        c2.wait()
        c3.wait()
        pltpu.sync_copy(n_v, n_out.at[pl.ds(base, b)])
        pltpu.sync_copy(t_v, t_out.at[pl.ds(base, b)])
        pltpu.sync_copy(e_v, e_out.at[pl.ds(base, b)])

    return k(ntab, ttab, emb, idx)


def _sc_gather_embed(emb, idx, chunk):
    n_idx = idx.shape[0]
    d = emb.shape[1]
    b = n_idx // _NW
    nchunks = b // chunk
    mesh = plsc.VectorSubcoreMesh(core_axis_name="c", subcore_axis_name="s")

    @functools.partial(
        pl.kernel,
        out_type=jax.ShapeDtypeStruct((n_idx, d), jnp.float32),
        mesh=mesh,
        scratch_types=[
            pltpu.VMEM((chunk,), jnp.int32),
            pltpu.VMEM((chunk, d), jnp.float32),
            pltpu.SemaphoreType.DMA,
        ],
    )
    def k(emb_hbm, idx_hbm, out_hbm, idx_v, rows_v, sem):
        wid = lax.axis_index("s") * 2 + lax.axis_index("c")
        base = wid * b
        for c in range(nchunks):
            off = base + c * chunk
            pltpu.sync_copy(idx_hbm.at[pl.ds(off, chunk)], idx_v)
            pltpu.async_copy(emb_hbm.at[idx_v], rows_v, sem).wait()
            pltpu.sync_copy(rows_v, out_hbm.at[pl.ds(off, chunk)])

    return k(emb, idx)


def _precise_cos(x):
    k = jnp.floor(x * 0.3183098861837907 + 0.5)
    y = ((x - k * 3.140625) - k * 0.0009676535846665502) \
        - k * 5.126565838509123e-12
    u = y * y
    acc = jnp.float32(-1.1470745597729725e-11)
    for c in (2.08767569878681e-09, -2.755731922398589e-07,
              2.48015873015873e-05, -1.3888888888888889e-03,
              4.1666666666666664e-02, -0.5, 1.0):
        acc = acc * u + c
    parity = k - 2.0 * jnp.floor(k * 0.5)
    return acc * (1.0 - 2.0 * parity)


def _attn_body(src_ref, seq_ref, t_ref, tn_ref, ngh_ref, freq_ref, phase_ref,
               wq_ref, wk_ref, wv_ref, wo_ref, mw1_ref, mb1_ref, mw2_ref,
               mb2_ref, out_ref):
    bq, d = src_ref.shape
    nn = tn_ref.shape[1]
    m = wq_ref.shape[0]
    n_head = 4
    d_head = m // n_head

    src = src_ref[...]
    phase = phase_ref[...]
    src_t = jnp.cos(phase)
    q_in = jnp.concatenate(
        [src, lax.broadcast_in_dim(src_t, (bq, d), (0, 1))], axis=1)
    q = jnp.dot(q_in, wq_ref[...], preferred_element_type=jnp.float32)

    delta = t_ref[...] - tn_ref[...]
    d3 = lax.broadcast_in_dim(delta, (bq, nn, d), (0, 1))
    freq3 = lax.broadcast_in_dim(freq_ref[...], (bq, nn, d), (1, 2))
    phase3 = lax.broadcast_in_dim(phase, (bq, nn, d), (1, 2))
    t_enc = _precise_cos(d3 * freq3 + phase3)
    seq = seq_ref[...].reshape(bq, nn, d)
    k_in = jnp.concatenate([seq, t_enc], axis=2).reshape(bq * nn, m)
    kk = jnp.dot(k_in, wk_ref[...], preferred_element_type=jnp.float32)
    vv = jnp.dot(k_in, wv_ref[...], preferred_element_type=jnp.float32)

    prod = lax.broadcast_in_dim(q, (bq, nn, m), (0, 2)) * kk.reshape(bq, nn, m)
    vv3 = vv.reshape(bq, nn, m)
    mask = ngh_ref[...] == 0
    scale = 1.0 / (d_head ** 0.5)
    o_heads = []
    for h in range(n_head):
        sh = jnp.sum(prod[:, :, h * d_head:(h + 1) * d_head], axis=2) * scale
        sh = jnp.where(mask, -1e10, sh)
        mh = jnp.max(sh, axis=1, keepdims=True)
        eh = jnp.exp(sh - mh)
        ah = eh / jnp.sum(eh, axis=1, keepdims=True)
        vh = vv3[:, :, h * d_head:(h + 1) * d_head]
        oh = jnp.sum(lax.broadcast_in_dim(ah, (bq, nn, d_head), (0, 1)) * vh,
                     axis=1)
        o_heads.append(oh)
    o = jnp.concatenate(o_heads, axis=1)
    o = jnp.dot(o, wo_ref[...], preferred_element_type=jnp.float32)

    h1 = jnp.concatenate([o, src], axis=1)
    h1 = jnp.maximum(
        jnp.dot(h1, mw1_ref[...], preferred_element_type=jnp.float32)
        + mb1_ref[...], 0.0)
    out_ref[...] = (jnp.dot(h1, mw2_ref[...], preferred_element_type=jnp.float32)
                    + mb2_ref[...])


def _attn_layer(src, seq_flat, t, tn, ngh, freq2, phase2, p, bq):
    nq, d = src.shape
    nn = tn.shape[1]
    m = 2 * d
    grid = (nq // bq,)
    full = lambda r, c: pl.BlockSpec((r, c), lambda i: (0, 0))
    return pl.pallas_call(
        _attn_body,
        grid=grid,
        in_specs=[
            pl.BlockSpec((bq, d), lambda i: (i, 0)),
            pl.BlockSpec((bq * nn, d), lambda i: (i, 0)),
            pl.BlockSpec((bq, 1), lambda i: (i, 0)),
            pl.BlockSpec((bq, nn), lambda i: (i, 0)),
            pl.BlockSpec((bq, nn), lambda i: (i, 0)),
            full(1, d),
            full(1, d),
            full(m, m),
            full(m, m),
            full(m, m),
            full(m, m),
            full(m + d, d),
            full(1, d),
            full(d, d),
            full(1, d),
        ],
        out_specs=pl.BlockSpec((bq, d), lambda i: (i, 0)),
        out_shape=jax.ShapeDtypeStruct((nq, d), jnp.float32),
    )(src, seq_flat, t, tn, ngh, freq2, phase2,
      p['Wq'], p['Wk'], p['Wv'], p['Wo'],
      p['mw1'], p['mb1'][None, :], p['mw2'], p['mb2'][None, :])


def _score_body(cs_ref, ct_ref, aw1_ref, ab1_ref, aw2_ref, ab2_ref, out_ref):
    hcat = jnp.concatenate([cs_ref[...], ct_ref[...]], axis=1)
    h = jnp.maximum(
        jnp.dot(hcat, aw1_ref[...], preferred_element_type=jnp.float32)
        + ab1_ref[...], 0.0)
    out_ref[...] = (jnp.dot(h, aw2_ref[...], preferred_element_type=jnp.float32)
                    + ab2_ref[...])


def _score_call(cs, ct, aw1, ab1, aw2, ab2):
    b, d = cs.shape
    aw2p = jnp.pad(aw2, ((0, 0), (0, d - aw2.shape[1])))
    ab2p = jnp.pad(ab2, (0, d - ab2.shape[0]))[None, :]
    out = pl.pallas_call(
        _score_body,
        out_shape=jax.ShapeDtypeStruct((b, d), jnp.float32),
    )(cs, ct, aw1, ab1[None, :], aw2p, ab2p)
    return out[:, 0]


def kernel(src_idx_l, target_idx_l, cut_time_l, num_neighbors, node_embed,
           ngh_node_table, ngh_time_table, time_freq, time_phase, params):
    nn = ngh_node_table.shape[1]

    q1 = jnp.concatenate([src_idx_l, target_idx_l])
    t1 = jnp.concatenate([cut_time_l, cut_time_l])

    n1, tn1, e_q1 = _sc_gather3(ngh_node_table, ngh_time_table, node_embed, q1)
    q2 = n1.reshape(-1)
    t2 = tn1.reshape(-1)
    n2, tn2, e_q2 = _sc_gather3(ngh_node_table, ngh_time_table, node_embed, q2)
    qk = n2.reshape(-1)
    ek2 = _sc_gather_embed(node_embed, qk, chunk=512)

    colpad = jnp.arange(nn)[None, :] >= num_neighbors
    n1m = jnp.where(colpad, 0, n1)
    n2m = jnp.where(colpad, 0, n2)

    freq2 = time_freq[None, :]
    phase2 = time_phase[None, :]

    c1a = _attn_layer(e_q1, e_q2, t1[:, None], tn1, n1m,
                      freq2, phase2, params['layer0'], bq=512)
    c1b = _attn_layer(e_q2, ek2, t2[:, None], tn2, n2m,
                      freq2, phase2, params['layer0'], bq=512)
    c2 = _attn_layer(c1a, c1b, t1[:, None], tn1, n1m,
                     freq2, phase2, params['layer1'], bq=512)

    b = src_idx_l.shape[0]
    return _score_call(c2[:b], c2[b:], params['aw1'], params['ab1'],
                       params['aw2'], params['ab2'])
